# baseline (speedup 1.0000x reference)
import functools

import jax
import jax.numpy as jnp
from jax import lax
from jax.experimental import pallas as pl
from jax.experimental.pallas import tpu as pltpu
from jax.experimental.pallas import tpu_sc as plsc

BATCH = 16384
NUM_FEAT = 32
L = 16

_info = plsc.get_sparse_core_info()
NC, NS = _info.num_cores, _info.num_subcores
NW = NC * NS
B_PER_W = BATCH // NW
CHUNK = 128
NCHUNK = B_PER_W // CHUNK


def _body(uidx_hbm, iidx_hbm, wU_hbm, wI_hbm, out_hbm,
          uidx_v, iidx_v, urows_v, irows_v, out_v, sem):
    wid = lax.axis_index("s") * NC + lax.axis_index("c")
    base = wid * B_PER_W

    for k in range(NCHUNK):
        pltpu.sync_copy(uidx_hbm.at[pl.ds(base + k * CHUNK, CHUNK)],
                        uidx_v.at[k])
        pltpu.sync_copy(iidx_hbm.at[pl.ds(base + k * CHUNK, CHUNK)],
                        iidx_v.at[k])

    copies = []
    for k in range(NCHUNK):
        copies.append(pltpu.async_copy(
            wU_hbm.at[uidx_v.at[k]],
            urows_v.at[pl.ds(k * CHUNK, CHUNK)], sem))
        copies.append(pltpu.async_copy(
            wI_hbm.at[iidx_v.at[k]],
            irows_v.at[pl.ds(k * CHUNK, CHUNK)], sem))
    for c in copies:
        c.wait()

    def group(g, _):
        row_ids = g * L + lax.iota(jnp.int32, L)
        acc = jnp.zeros((L,), jnp.float32)
        for j in range(NUM_FEAT):
            col = jnp.full((L,), j, jnp.int32)
            u = plsc.load_gather(urows_v, [row_ids, col])
            v = plsc.load_gather(irows_v, [row_ids, col])
            acc = acc + u * v
        p = 1.0 / (1.0 + jnp.exp(-acc))
        plsc.store_scatter(out_v, [row_ids], p)
        return 0

    lax.fori_loop(0, B_PER_W // L, group, 0)

    pltpu.sync_copy(out_v, out_hbm.at[pl.ds(base, B_PER_W)])


@functools.cache
def _build():
    mesh = plsc.VectorSubcoreMesh(core_axis_name="c", subcore_axis_name="s")
    return pl.kernel(
        _body,
        mesh=mesh,
        compiler_params=pltpu.CompilerParams(use_tc_tiling_on_sc=False,
                                             needs_layout_passes=False),
        out_type=jax.ShapeDtypeStruct((BATCH,), jnp.float32),
        scratch_types=[
            pltpu.VMEM((NCHUNK, CHUNK), jnp.int32),
            pltpu.VMEM((NCHUNK, CHUNK), jnp.int32),
            pltpu.VMEM((B_PER_W, NUM_FEAT), jnp.float32),
            pltpu.VMEM((B_PER_W, NUM_FEAT), jnp.float32),
            pltpu.VMEM((B_PER_W,), jnp.float32),
            pltpu.SemaphoreType.DMA,
        ],
    )


def kernel(user_indices, item_indices, w_User, w_Item):
    return _build()(user_indices.astype(jnp.int32),
                    item_indices.astype(jnp.int32),
                    w_User, w_Item)
